# baseline (speedup 1.0000x reference)
import jax
import jax.numpy as jnp
from jax import lax
from jax.experimental import pallas as pl
from jax.experimental.pallas import tpu as pltpu
from jax.experimental.pallas import tpu_sc as plsc

_DIM = 64
_PADW = 128
_B = 16384
_L = 20
_EPS = 1e-5

_NWORKERS = 32
_XROWS_PER_W = _B // _NWORKERS
_NX = 16
_CHUNK = _NX * _L
_NCHUNKS = _XROWS_PER_W // _NX
_UNROLL = 8


def _rsqrt16(v):
    vi = plsc.bitcast(v, jnp.int32)
    yi = jnp.int32(0x5F3759DF) - lax.shift_right_arithmetic(vi, jnp.int32(1))
    y = plsc.bitcast(yi, jnp.float32)
    half_v = v * 0.5
    for _ in range(2):
        y = y * (1.5 - half_v * y * y)
    return y


def _sc_body(x_hbm, table_hbm, gamma_hbm, beta_hbm, out_hbm,
             idx2, rb, ob, gamma_v, beta_v, sem_i, sem_g, sem_o):
    cid = lax.axis_index("c")
    sid = lax.axis_index("s")
    wid = sid * 2 + cid

    pltpu.sync_copy(gamma_hbm, gamma_v)
    pltpu.sync_copy(beta_hbm, beta_v)
    base_xr = wid * _XROWS_PER_W
    base_row = base_xr * _L

    gs = [gamma_v[pl.ds(k * 16, 16)] for k in range(4)]
    bs = [beta_v[pl.ds(k * 16, 16)] for k in range(4)]
    lane = lax.iota(jnp.int32, 16)
    bfly = [jnp.bitwise_xor(lane, jnp.int32(m)) for m in (1, 2, 4, 8)]

    def idx_desc(ci):
        return pltpu.make_async_copy(
            x_hbm.at[pl.ds(base_xr + ci * _NX, _NX)],
            idx2.at[lax.rem(ci, 2)],
            sem_i,
        )

    def gather_descs(ci):
        slot = lax.rem(ci, 2)
        return [
            pltpu.make_async_copy(
                table_hbm.at[idx2.at[slot, r]],
                rb.at[slot, pl.ds(r * _L, _L)],
                sem_g,
            )
            for r in range(_NX)
        ]

    def out_desc(ci):
        return pltpu.make_async_copy(
            ob.at[lax.rem(ci, 2)],
            out_hbm.at[pl.ds(base_row + ci * _CHUNK, _CHUNK)],
            sem_o,
        )

    idx_desc(0).start()
    idx_desc(0).wait()
    for d in gather_descs(0):
        d.start()
    idx_desc(1).start()

    def chunk_body(ci, carry):
        @pl.when(ci + 1 < _NCHUNKS)
        def _():
            idx_desc(ci + 1).wait()
            for d in gather_descs(ci + 1):
                d.start()

        @pl.when(ci + 2 < _NCHUNKS)
        def _():
            idx_desc(ci + 2).start()

        for d in gather_descs(ci):
            d.wait()

        @pl.when(ci >= 2)
        def _():
            out_desc(ci - 2).wait()

        slot = lax.rem(ci, 2)
        buf = rb.at[slot]
        obuf = ob.at[slot]

        def rows_body(it, c2):
            for u in range(_UNROLL):
                row = it * _UNROLL + u
                v = [buf[row, pl.ds(k * 16, 16)] for k in range(4)]
                s = (v[0] + v[1]) + (v[2] + v[3])
                q = ((v[0] * v[0] + v[1] * v[1])
                     + (v[2] * v[2] + v[3] * v[3]))
                for bf in bfly:
                    s = s + s.at[bf].get(mode="promise_in_bounds")
                    q = q + q.at[bf].get(mode="promise_in_bounds")
                mean = s * (1.0 / _DIM)
                var = q * (1.0 / _DIM) - mean * mean
                inv = _rsqrt16(var + _EPS)
                shift = -mean * inv
                for k in range(4):
                    obuf[row, pl.ds(k * 16, 16)] = (
                        (v[k] * inv + shift) * gs[k] + bs[k])
            return c2

        lax.fori_loop(0, _CHUNK // _UNROLL, rows_body, 0)

        out_desc(ci).start()
        return carry

    lax.fori_loop(0, _NCHUNKS, chunk_body, 0)
    out_desc(_NCHUNKS - 2).wait()
    out_desc(_NCHUNKS - 1).wait()


@jax.jit
def kernel(x, table, gamma, beta):
    xp = x.astype(jnp.int32)
    tp = jnp.pad(table, ((0, 0), (0, _PADW - _DIM)))
    mesh = plsc.VectorSubcoreMesh(core_axis_name="c", subcore_axis_name="s")
    out = pl.kernel(
        _sc_body,
        out_type=jax.ShapeDtypeStruct((_B * _L, _DIM), jnp.float32),
        mesh=mesh,
        compiler_params=pltpu.CompilerParams(
            needs_layout_passes=False, use_tc_tiling_on_sc=False),
        scratch_types=[
            pltpu.VMEM((2, _NX, _L), jnp.int32),
            pltpu.VMEM((2, _CHUNK, _PADW), jnp.float32),
            pltpu.VMEM((2, _CHUNK, _DIM), jnp.float32),
            pltpu.VMEM((_DIM,), jnp.float32),
            pltpu.VMEM((_DIM,), jnp.float32),
            pltpu.SemaphoreType.DMA,
            pltpu.SemaphoreType.DMA,
            pltpu.SemaphoreType.DMA,
        ],
    )(xp, tp, gamma, beta)
    return out.reshape(_B, _L, _DIM)


# speedup vs baseline: 1.0168x; 1.0168x over previous
import jax
import jax.numpy as jnp
from jax import lax
from jax.experimental import pallas as pl
from jax.experimental.pallas import tpu as pltpu
from jax.experimental.pallas import tpu_sc as plsc

_DIM = 64
_PADW = 128
_B = 16384
_L = 20
_EPS = 1e-5

_NWORKERS = 32
_XROWS_PER_W = _B // _NWORKERS
_NX = 16
_CHUNK = _NX * _L
_NCHUNKS = _XROWS_PER_W // _NX
_UNROLL = 8


def _rsqrt16(v):
    vi = plsc.bitcast(v, jnp.int32)
    yi = jnp.int32(0x5F3759DF) - lax.shift_right_arithmetic(vi, jnp.int32(1))
    y = plsc.bitcast(yi, jnp.float32)
    half_v = v * 0.5
    for _ in range(2):
        y = y * (1.5 - half_v * y * y)
    return y


def _sc_body(x_hbm, table_hbm, gamma_hbm, beta_hbm, out_hbm,
             idx2, rb, ob, gamma_v, beta_v, sem_i, sem_g, sem_o):
    cid = lax.axis_index("c")
    sid = lax.axis_index("s")
    wid = sid * 2 + cid

    pltpu.sync_copy(gamma_hbm, gamma_v)
    pltpu.sync_copy(beta_hbm, beta_v)
    base_xr = wid * _XROWS_PER_W
    base_row = base_xr * _L

    gs = [gamma_v[pl.ds(k * 16, 16)] for k in range(4)]
    bs = [beta_v[pl.ds(k * 16, 16)] for k in range(4)]
    last = jnp.full((16,), 15, jnp.int32)

    def idx_desc(ci):
        return pltpu.make_async_copy(
            x_hbm.at[pl.ds(base_xr + ci * _NX, _NX)],
            idx2.at[lax.rem(ci, 2)],
            sem_i,
        )

    def gather_descs(ci):
        slot = lax.rem(ci, 2)
        return [
            pltpu.make_async_copy(
                table_hbm.at[idx2.at[slot, r]],
                rb.at[slot, pl.ds(r * _L, _L)],
                sem_g,
            )
            for r in range(_NX)
        ]

    def out_desc(ci):
        return pltpu.make_async_copy(
            ob.at[lax.rem(ci, 2)],
            out_hbm.at[pl.ds(base_row + ci * _CHUNK, _CHUNK)],
            sem_o,
        )

    idx_desc(0).start()
    idx_desc(0).wait()
    for d in gather_descs(0):
        d.start()
    idx_desc(1).start()

    def chunk_body(ci, carry):
        @pl.when(ci + 1 < _NCHUNKS)
        def _():
            idx_desc(ci + 1).wait()
            for d in gather_descs(ci + 1):
                d.start()

        @pl.when(ci + 2 < _NCHUNKS)
        def _():
            idx_desc(ci + 2).start()

        for d in gather_descs(ci):
            d.wait()

        @pl.when(ci >= 2)
        def _():
            out_desc(ci - 2).wait()

        slot = lax.rem(ci, 2)
        buf = rb.at[slot]
        obuf = ob.at[slot]

        def rows_body(it, c2):
            for u in range(_UNROLL):
                row = it * _UNROLL + u
                v = [buf[row, pl.ds(k * 16, 16)] for k in range(4)]
                s = (v[0] + v[1]) + (v[2] + v[3])
                q = ((v[0] * v[0] + v[1] * v[1])
                     + (v[2] * v[2] + v[3] * v[3]))
                tot = plsc.cumsum(s).at[last].get(mode="promise_in_bounds")
                totq = plsc.cumsum(q).at[last].get(mode="promise_in_bounds")
                mean = tot * (1.0 / _DIM)
                var = totq * (1.0 / _DIM) - mean * mean
                inv = _rsqrt16(var + _EPS)
                shift = -mean * inv
                for k in range(4):
                    obuf[row, pl.ds(k * 16, 16)] = (
                        (v[k] * inv + shift) * gs[k] + bs[k])
            return c2

        lax.fori_loop(0, _CHUNK // _UNROLL, rows_body, 0)

        out_desc(ci).start()
        return carry

    lax.fori_loop(0, _NCHUNKS, chunk_body, 0)
    out_desc(_NCHUNKS - 2).wait()
    out_desc(_NCHUNKS - 1).wait()


@jax.jit
def kernel(x, table, gamma, beta):
    xp = x.astype(jnp.int32)
    tp = jnp.pad(table, ((0, 0), (0, _PADW - _DIM)))
    mesh = plsc.VectorSubcoreMesh(core_axis_name="c", subcore_axis_name="s")
    out = pl.kernel(
        _sc_body,
        out_type=jax.ShapeDtypeStruct((_B * _L, _DIM), jnp.float32),
        mesh=mesh,
        compiler_params=pltpu.CompilerParams(
            needs_layout_passes=False, use_tc_tiling_on_sc=False),
        scratch_types=[
            pltpu.VMEM((2, _NX, _L), jnp.int32),
            pltpu.VMEM((2, _CHUNK, _PADW), jnp.float32),
            pltpu.VMEM((2, _CHUNK, _DIM), jnp.float32),
            pltpu.VMEM((_DIM,), jnp.float32),
            pltpu.VMEM((_DIM,), jnp.float32),
            pltpu.SemaphoreType.DMA,
            pltpu.SemaphoreType.DMA,
            pltpu.SemaphoreType.DMA,
        ],
    )(xp, tp, gamma, beta)
    return out.reshape(_B, _L, _DIM)


# speedup vs baseline: 1.0171x; 1.0003x over previous
import jax
import jax.numpy as jnp
from jax import lax
from jax.experimental import pallas as pl
from jax.experimental.pallas import tpu as pltpu
from jax.experimental.pallas import tpu_sc as plsc

_DIM = 64
_PADW = 128
_B = 16384
_L = 20
_EPS = 1e-5

_NWORKERS = 32
_XROWS_PER_W = _B // _NWORKERS
_NX = 16
_CHUNK = _NX * _L
_NCHUNKS = _XROWS_PER_W // _NX
_UNROLL = 8


def _rsqrt16(v):
    vi = plsc.bitcast(v, jnp.int32)
    yi = jnp.int32(0x5F3759DF) - lax.shift_right_arithmetic(vi, jnp.int32(1))
    y = plsc.bitcast(yi, jnp.float32)
    half_v = v * 0.5
    y = y * (1.5 - half_v * y * y)
    return y * (1.5 - half_v * y * y)


def _sc_body(x_hbm, table_hbm, gamma_hbm, beta_hbm, out_hbm,
             idx2, rb, ob, gamma_v, beta_v, sem_i, sem_g, sem_o):
    cid = lax.axis_index("c")
    sid = lax.axis_index("s")
    wid = sid * 2 + cid

    pltpu.sync_copy(gamma_hbm, gamma_v)
    pltpu.sync_copy(beta_hbm, beta_v)
    base_xr = wid * _XROWS_PER_W
    base_row = base_xr * _L

    gs = [gamma_v[pl.ds(k * 16, 16)] for k in range(4)]
    bs = [beta_v[pl.ds(k * 16, 16)] for k in range(4)]
    last = jnp.full((16,), 15, jnp.int32)

    def idx_desc(ci):
        return pltpu.make_async_copy(
            x_hbm.at[pl.ds(base_xr + ci * _NX, _NX)],
            idx2.at[lax.rem(ci, 2)],
            sem_i,
        )

    def gather_descs(ci):
        slot = lax.rem(ci, 2)
        return [
            pltpu.make_async_copy(
                table_hbm.at[idx2.at[slot, r]],
                rb.at[slot, pl.ds(r * _L, _L)],
                sem_g,
            )
            for r in range(_NX)
        ]

    def out_desc(ci):
        return pltpu.make_async_copy(
            ob.at[lax.rem(ci, 2)],
            out_hbm.at[pl.ds(base_row + ci * _CHUNK, _CHUNK)],
            sem_o,
        )

    idx_desc(0).start()
    idx_desc(0).wait()
    for d in gather_descs(0):
        d.start()
    idx_desc(1).start()

    def chunk_body(ci, carry):
        @pl.when(ci + 1 < _NCHUNKS)
        def _():
            idx_desc(ci + 1).wait()
            for d in gather_descs(ci + 1):
                d.start()

        @pl.when(ci + 2 < _NCHUNKS)
        def _():
            idx_desc(ci + 2).start()

        for d in gather_descs(ci):
            d.wait()

        @pl.when(ci >= 2)
        def _():
            out_desc(ci - 2).wait()

        slot = lax.rem(ci, 2)
        buf = rb.at[slot]
        obuf = ob.at[slot]

        def rows_body(it, c2):
            for u in range(_UNROLL):
                row = it * _UNROLL + u
                v = [buf[row, pl.ds(k * 16, 16)] for k in range(4)]
                s = (v[0] + v[1]) + (v[2] + v[3])
                q = ((v[0] * v[0] + v[1] * v[1])
                     + (v[2] * v[2] + v[3] * v[3]))
                tot = plsc.cumsum(s).at[last].get(mode="promise_in_bounds")
                totq = plsc.cumsum(q).at[last].get(mode="promise_in_bounds")
                mean = tot * (1.0 / _DIM)
                var = totq * (1.0 / _DIM) - mean * mean
                inv = _rsqrt16(var + _EPS)
                shift = -mean * inv
                for k in range(4):
                    obuf[row, pl.ds(k * 16, 16)] = (
                        (v[k] * inv + shift) * gs[k] + bs[k])
            return c2

        lax.fori_loop(0, _CHUNK // _UNROLL, rows_body, 0)

        out_desc(ci).start()
        return carry

    lax.fori_loop(0, _NCHUNKS, chunk_body, 0)
    out_desc(_NCHUNKS - 2).wait()
    out_desc(_NCHUNKS - 1).wait()


@jax.jit
def kernel(x, table, gamma, beta):
    xp = x.astype(jnp.int32)
    tp = jnp.pad(table, ((0, 0), (0, _PADW - _DIM)))
    mesh = plsc.VectorSubcoreMesh(core_axis_name="c", subcore_axis_name="s")
    out = pl.kernel(
        _sc_body,
        out_type=jax.ShapeDtypeStruct((_B * _L, _DIM), jnp.float32),
        mesh=mesh,
        compiler_params=pltpu.CompilerParams(
            needs_layout_passes=False, use_tc_tiling_on_sc=False),
        scratch_types=[
            pltpu.VMEM((2, _NX, _L), jnp.int32),
            pltpu.VMEM((2, _CHUNK, _PADW), jnp.float32),
            pltpu.VMEM((2, _CHUNK, _DIM), jnp.float32),
            pltpu.VMEM((_DIM,), jnp.float32),
            pltpu.VMEM((_DIM,), jnp.float32),
            pltpu.SemaphoreType.DMA,
            pltpu.SemaphoreType.DMA,
            pltpu.SemaphoreType.DMA,
        ],
    )(xp, tp, gamma, beta)
    return out.reshape(_B, _L, _DIM)
